# baseline (speedup 1.0000x reference)
import functools

import jax
import jax.numpy as jnp
from jax import lax
from jax.experimental import pallas as pl
from jax.experimental.pallas import tpu as pltpu
from jax.experimental.pallas import tpu_sc as plsc

_NC = 2
_NS = 16
_NW = _NC * _NS
_NB = 8
_A = 6


def _gather_body(ch, seq, embed, idx_hbm, table_hbm, out_hbm, idx_v, rows_v,
                 gsem, wsem):
    wid = lax.axis_index("s") * _NC + lax.axis_index("c")
    pltpu.sync_copy(idx_hbm.at[wid], idx_v)
    base = wid * ch

    def gather(c, b):
        return pltpu.make_async_copy(
            table_hbm.at[idx_v.at[c]], rows_v.at[b], gsem.at[b])

    def write(c, b):
        return pltpu.make_async_copy(
            rows_v.at[b], out_hbm.at[base + c], wsem.at[b])

    def step(c, b, do_wait_w, do_gather):
        gather(c, b).wait()
        write(c, b).start()
        f = c + _A
        bf = (b + _A) % _NB
        if do_wait_w:
            write(f - _NB, bf).wait()
        if do_gather:
            gather(f, bf).start()

    for r in range(_A):
        gather(r, r % _NB).start()

    for r in range(_NB):
        step(r, r, do_wait_w=(r + _A >= _NB), do_gather=True)

    def outer(j, carry):
        c0 = j * _NB
        for r in range(_NB):
            step(c0 + r, r, do_wait_w=True, do_gather=True)
        return carry

    lax.fori_loop(1, ch // _NB - 1, outer, 0)

    for r in range(_NB):
        step(ch - _NB + r, r, do_wait_w=(r + _A < _NB),
             do_gather=(r + _A < _NB))

    for b in range(_NB):
        write(ch - _NB + b, b).wait()


@functools.partial(jax.jit, static_argnums=(2, 3, 4))
def _sc_gather(idx, table, ch, seq, embed):
    mesh = plsc.VectorSubcoreMesh(core_axis_name="c", subcore_axis_name="s")
    fn = pl.kernel(
        functools.partial(_gather_body, ch, seq, embed),
        mesh=mesh,
        out_type=jax.ShapeDtypeStruct((_NW * ch, seq, embed), jnp.float32),
        scratch_types=[
            pltpu.VMEM((ch, seq), jnp.int32),
            pltpu.VMEM((_NB, seq, embed), jnp.float32),
            pltpu.SemaphoreType.DMA((_NB,)),
            pltpu.SemaphoreType.DMA((_NB,)),
        ],
    )
    return fn(idx, table)


_K = 2


def kernel(X, embedding_weight):
    b, s = X.shape
    vocab, embed = embedding_weight.shape
    bk = b // _K
    ch = bk // _NW
    outs = []
    for k in range(_K):
        idx = X[k * bk:(k + 1) * bk].reshape(_NW, ch, s).astype(jnp.int32)
        outs.append(_sc_gather(idx, embedding_weight, ch, s, embed))
    return jnp.concatenate(outs, axis=0)
